# baseline (device time: 163271 ns/iter reference)
import jax
import jax.numpy as jnp
from jax import lax
from jax.experimental import pallas as pl
from jax.experimental.pallas import tpu as pltpu

N_DEV = 4


def kernel(A, B):
    m, k = A.shape
    k2, n = B.shape
    assert k == k2
    m_per = m // N_DEV

    def body(a_ref, b_ref, out_ref, part_ref, comm_ref, send_sems, recv_sems):
        my = lax.axis_index("i")
        left = (my - 1) % N_DEV
        right = (my + 1) % N_DEV

        barrier_sem = pltpu.get_barrier_semaphore()
        for nbr in [left, right]:
            pl.semaphore_signal(
                barrier_sem, inc=1,
                device_id=(nbr,), device_id_type=pl.DeviceIdType.MESH,
            )
        pl.semaphore_wait(barrier_sem, 2)

        part_ref[:, :] = jnp.dot(
            a_ref[:, :], b_ref[:, :], preferred_element_type=jnp.float32
        )

        c0 = (my - 1) % N_DEV
        comm_ref[0, :, :] = part_ref[pl.ds(c0 * m_per, m_per), :]

        for h in range(N_DEV - 1):
            send_slot = h % 2
            recv_slot = (h + 1) % 2
            rdma = pltpu.make_async_remote_copy(
                src_ref=comm_ref.at[send_slot],
                dst_ref=comm_ref.at[recv_slot],
                send_sem=send_sems.at[send_slot],
                recv_sem=recv_sems.at[recv_slot],
                device_id=(right,),
                device_id_type=pl.DeviceIdType.MESH,
            )
            rdma.start()
            rdma.wait()

            c_recv = (my - h - 2) % N_DEV
            comm_ref[recv_slot, :, :] = (
                comm_ref[recv_slot, :, :]
                + part_ref[pl.ds(c_recv * m_per, m_per), :]
            )

        out_ref[:, :] = comm_ref[(N_DEV - 1) % 2, :, :]

    return pl.pallas_call(
        body,
        out_shape=jax.ShapeDtypeStruct((m_per, n), jnp.float32),
        in_specs=[
            pl.BlockSpec(memory_space=pltpu.VMEM),
            pl.BlockSpec(memory_space=pltpu.VMEM),
        ],
        out_specs=pl.BlockSpec(memory_space=pltpu.VMEM),
        scratch_shapes=[
            pltpu.VMEM((m, n), jnp.float32),
            pltpu.VMEM((2, m_per, n), jnp.float32),
            pltpu.SemaphoreType.DMA((2,)),
            pltpu.SemaphoreType.DMA((2,)),
        ],
        compiler_params=pltpu.CompilerParams(collective_id=0),
    )(A, B)


# device time: 91208 ns/iter; 1.7901x vs baseline; 1.7901x over previous
import jax
import jax.numpy as jnp
from jax import lax
from jax.experimental import pallas as pl
from jax.experimental.pallas import tpu as pltpu

N_DEV = 4


def kernel(A, B):
    m, k = A.shape
    k2, n = B.shape
    assert k == k2
    m_per = m // N_DEV
    h_per = m_per // 2

    def body(a_ref, b_ref, out_ref, part_ref, cw_ref, ccw_ref,
             cw_send, cw_recv, ccw_send, ccw_recv):
        my = lax.axis_index("i")
        left = (my + N_DEV - 1) % N_DEV
        right = (my + 1) % N_DEV

        barrier_sem = pltpu.get_barrier_semaphore()
        for nbr in [left, right]:
            pl.semaphore_signal(
                barrier_sem, inc=1,
                device_id=(nbr,), device_id_type=pl.DeviceIdType.MESH,
            )
        pl.semaphore_wait(barrier_sem, 2)

        def compute_chunk(c):
            part_ref[pl.ds(c * m_per, m_per), :] = jnp.dot(
                a_ref[pl.ds(c * m_per, m_per), :], b_ref[:, :],
                preferred_element_type=jnp.float32,
            )

        c_cw = (my + N_DEV - 1) % N_DEV
        c_ccw = (my + 1) % N_DEV
        compute_chunk(c_cw)
        compute_chunk(c_ccw)
        cw_ref[0, :, :] = part_ref[pl.ds(c_cw * m_per, h_per), :]
        ccw_ref[0, :, :] = part_ref[pl.ds(c_ccw * m_per + h_per, h_per), :]

        for h in range(N_DEV - 1):
            s = h % 2
            r = (h + 1) % 2
            cw = pltpu.make_async_remote_copy(
                src_ref=cw_ref.at[s], dst_ref=cw_ref.at[r],
                send_sem=cw_send.at[s], recv_sem=cw_recv.at[r],
                device_id=(right,), device_id_type=pl.DeviceIdType.MESH,
            )
            ccw = pltpu.make_async_remote_copy(
                src_ref=ccw_ref.at[s], dst_ref=ccw_ref.at[r],
                send_sem=ccw_send.at[s], recv_sem=ccw_recv.at[r],
                device_id=(left,), device_id_type=pl.DeviceIdType.MESH,
            )
            cw.start()
            ccw.start()

            if h == 0:
                compute_chunk((my + 2) % N_DEV)
                compute_chunk(my)

            cw.wait()
            ccw.wait()

            rc_cw = (my + N_DEV - h - 2) % N_DEV
            rc_ccw = (my + h + 2) % N_DEV
            if h < N_DEV - 2:
                cw_ref[r, :, :] = (
                    cw_ref[r, :, :]
                    + part_ref[pl.ds(rc_cw * m_per, h_per), :]
                )
                ccw_ref[r, :, :] = (
                    ccw_ref[r, :, :]
                    + part_ref[pl.ds(rc_ccw * m_per + h_per, h_per), :]
                )
            else:
                out_ref[0:h_per, :] = (
                    cw_ref[r, :, :] + part_ref[pl.ds(my * m_per, h_per), :]
                )
                out_ref[h_per:m_per, :] = (
                    ccw_ref[r, :, :]
                    + part_ref[pl.ds(my * m_per + h_per, h_per), :]
                )

    return pl.pallas_call(
        body,
        out_shape=jax.ShapeDtypeStruct((m_per, n), jnp.float32),
        in_specs=[
            pl.BlockSpec(memory_space=pltpu.VMEM),
            pl.BlockSpec(memory_space=pltpu.VMEM),
        ],
        out_specs=pl.BlockSpec(memory_space=pltpu.VMEM),
        scratch_shapes=[
            pltpu.VMEM((m, n), jnp.float32),
            pltpu.VMEM((2, h_per, n), jnp.float32),
            pltpu.VMEM((2, h_per, n), jnp.float32),
            pltpu.SemaphoreType.DMA((2,)),
            pltpu.SemaphoreType.DMA((2,)),
            pltpu.SemaphoreType.DMA((2,)),
            pltpu.SemaphoreType.DMA((2,)),
        ],
        compiler_params=pltpu.CompilerParams(collective_id=0),
    )(A, B)


# device time: 88466 ns/iter; 1.8456x vs baseline; 1.0310x over previous
import jax
import jax.numpy as jnp
from jax import lax
from jax.experimental import pallas as pl
from jax.experimental.pallas import tpu as pltpu

N_DEV = 4


def kernel(A, B):
    m, k = A.shape
    k2, n = B.shape
    assert k == k2
    m_per = m // N_DEV
    h_per = m_per // 2

    def body(a_ref, b_ref, out_ref, part_ref, cw_ref, ccw_ref,
             cw_send, cw_recv, ccw_send, ccw_recv):
        my = lax.axis_index("i")
        left = (my + N_DEV - 1) % N_DEV
        right = (my + 1) % N_DEV

        barrier_sem = pltpu.get_barrier_semaphore()
        for nbr in [left, right]:
            pl.semaphore_signal(
                barrier_sem, inc=1,
                device_id=(nbr,), device_id_type=pl.DeviceIdType.MESH,
            )
        pl.semaphore_wait(barrier_sem, 2)

        def compute_chunk(c):
            part_ref[pl.ds(c * m_per, m_per), :] = jnp.dot(
                a_ref[pl.ds(c * m_per, m_per), :], b_ref[:, :],
                preferred_element_type=jnp.float32,
            )

        def compute_half(c, half):
            start = c * m_per + half * h_per
            part_ref[pl.ds(start, h_per), :] = jnp.dot(
                a_ref[pl.ds(start, h_per), :], b_ref[:, :],
                preferred_element_type=jnp.float32,
            )

        c_cw = (my + N_DEV - 1) % N_DEV
        c_ccw = (my + 1) % N_DEV
        cw_ref[0, :, :] = jnp.dot(
            a_ref[pl.ds(c_cw * m_per, h_per), :], b_ref[:, :],
            preferred_element_type=jnp.float32,
        )
        ccw_ref[0, :, :] = jnp.dot(
            a_ref[pl.ds(c_ccw * m_per + h_per, h_per), :], b_ref[:, :],
            preferred_element_type=jnp.float32,
        )

        for h in range(N_DEV - 1):
            s = h % 2
            r = (h + 1) % 2
            cw = pltpu.make_async_remote_copy(
                src_ref=cw_ref.at[s], dst_ref=cw_ref.at[r],
                send_sem=cw_send.at[s], recv_sem=cw_recv.at[r],
                device_id=(right,), device_id_type=pl.DeviceIdType.MESH,
            )
            ccw = pltpu.make_async_remote_copy(
                src_ref=ccw_ref.at[s], dst_ref=ccw_ref.at[r],
                send_sem=ccw_send.at[s], recv_sem=ccw_recv.at[r],
                device_id=(left,), device_id_type=pl.DeviceIdType.MESH,
            )
            cw.start()
            ccw.start()

            if h == 0:
                compute_chunk((my + 2) % N_DEV)
                compute_half(c_ccw, 0)
                compute_half(c_cw, 1)
            elif h == 1:
                compute_chunk(my)

            cw.wait()
            ccw.wait()

            rc_cw = (my + N_DEV - h - 2) % N_DEV
            rc_ccw = (my + h + 2) % N_DEV
            if h < N_DEV - 2:
                cw_ref[r, :, :] = (
                    cw_ref[r, :, :]
                    + part_ref[pl.ds(rc_cw * m_per, h_per), :]
                )
                ccw_ref[r, :, :] = (
                    ccw_ref[r, :, :]
                    + part_ref[pl.ds(rc_ccw * m_per + h_per, h_per), :]
                )
            else:
                out_ref[0:h_per, :] = (
                    cw_ref[r, :, :] + part_ref[pl.ds(my * m_per, h_per), :]
                )
                out_ref[h_per:m_per, :] = (
                    ccw_ref[r, :, :]
                    + part_ref[pl.ds(my * m_per + h_per, h_per), :]
                )

    return pl.pallas_call(
        body,
        out_shape=jax.ShapeDtypeStruct((m_per, n), jnp.float32),
        in_specs=[
            pl.BlockSpec(memory_space=pltpu.VMEM),
            pl.BlockSpec(memory_space=pltpu.VMEM),
        ],
        out_specs=pl.BlockSpec(memory_space=pltpu.VMEM),
        scratch_shapes=[
            pltpu.VMEM((m, n), jnp.float32),
            pltpu.VMEM((2, h_per, n), jnp.float32),
            pltpu.VMEM((2, h_per, n), jnp.float32),
            pltpu.SemaphoreType.DMA((2,)),
            pltpu.SemaphoreType.DMA((2,)),
            pltpu.SemaphoreType.DMA((2,)),
            pltpu.SemaphoreType.DMA((2,)),
        ],
        compiler_params=pltpu.CompilerParams(collective_id=0),
    )(A, B)


# device time: 83083 ns/iter; 1.9652x vs baseline; 1.0648x over previous
import jax
import jax.numpy as jnp
from jax import lax
from jax.experimental import pallas as pl
from jax.experimental.pallas import tpu as pltpu

N_DEV = 4
NSUB = 2
R = 2 * NSUB


def kernel(A, B):
    m, k = A.shape
    k2, n = B.shape
    assert k == k2
    m_per = m // N_DEV
    h_per = m_per // 2
    rpr = m_per // R

    def body(a_ref, b_ref, out_ref, part_ref, bufs, send_sems, recv_sems):
        my = lax.axis_index("i")
        left = (my + N_DEV - 1) % N_DEV
        right = (my + 1) % N_DEV

        barrier_sem = pltpu.get_barrier_semaphore()
        for nbr in [left, right]:
            pl.semaphore_signal(
                barrier_sem, inc=1,
                device_id=(nbr,), device_id_type=pl.DeviceIdType.MESH,
            )
        pl.semaphore_wait(barrier_sem, 2)

        def ring_offset(j):
            return (0 if j % 2 == 0 else h_per) + (j // 2) * rpr

        def send_chunk(j, h):
            if j % 2 == 0:
                return (my + N_DEV - h - 1) % N_DEV
            return (my + h + 1) % N_DEV

        def recv_chunk(j, h):
            if j % 2 == 0:
                return (my + N_DEV - h - 2) % N_DEV
            return (my + h + 2) % N_DEV

        def make_rdma(j, h):
            s = h % 2
            r = (h + 1) % 2
            tgt = right if j % 2 == 0 else left
            return pltpu.make_async_remote_copy(
                src_ref=bufs.at[j, s], dst_ref=bufs.at[j, r],
                send_sem=send_sems.at[j, s], recv_sem=recv_sems.at[j, r],
                device_id=(tgt,), device_id_type=pl.DeviceIdType.MESH,
            )

        def part_rows(c, off, nrows):
            return part_ref[pl.ds(c * m_per + off, nrows), :]

        def compute_rows(start, nrows):
            part_ref[pl.ds(start, nrows), :] = jnp.dot(
                a_ref[pl.ds(start, nrows), :], b_ref[:, :],
                preferred_element_type=jnp.float32,
            )

        descs = {}
        for j in range(R):
            off = ring_offset(j)
            c0 = send_chunk(j, 0)
            bufs[j, 0] = jnp.dot(
                a_ref[pl.ds(c0 * m_per + off, rpr), :], b_ref[:, :],
                preferred_element_type=jnp.float32,
            )
            descs[(j, 0)] = make_rdma(j, 0)
            descs[(j, 0)].start()

        compute_rows(((my + 2) % N_DEV) * m_per, m_per)
        compute_rows(((my + 1) % N_DEV) * m_per, h_per)
        compute_rows(((my + N_DEV - 1) % N_DEV) * m_per + h_per, h_per)

        for h in range(N_DEV - 1):
            r = (h + 1) % 2
            for j in range(R):
                off = ring_offset(j)
                descs[(j, h)].wait()
                if h < N_DEV - 2:
                    rc = recv_chunk(j, h)
                    bufs[j, r] = bufs[j, r] + part_rows(rc, off, rpr)
                    descs[(j, h + 1)] = make_rdma(j, h + 1)
                    descs[(j, h + 1)].start()
                    if h == 1:
                        compute_rows(my * m_per + off, rpr)
                else:
                    out_ref[pl.ds(off, rpr), :] = (
                        bufs[j, r] + part_rows(my, off, rpr)
                    )

    return pl.pallas_call(
        body,
        out_shape=jax.ShapeDtypeStruct((m_per, n), jnp.float32),
        in_specs=[
            pl.BlockSpec(memory_space=pltpu.VMEM),
            pl.BlockSpec(memory_space=pltpu.VMEM),
        ],
        out_specs=pl.BlockSpec(memory_space=pltpu.VMEM),
        scratch_shapes=[
            pltpu.VMEM((m, n), jnp.float32),
            pltpu.VMEM((R, 2, rpr, n), jnp.float32),
            pltpu.SemaphoreType.DMA((R, 2)),
            pltpu.SemaphoreType.DMA((R, 2)),
        ],
        compiler_params=pltpu.CompilerParams(collective_id=0),
    )(A, B)
